# baseline (speedup 1.0000x reference)
import functools

import jax
import jax.numpy as jnp
from jax import lax
from jax.experimental import pallas as pl
from jax.experimental.pallas import tpu as pltpu
from jax.experimental.pallas import tpu_sc as plsc

_TIME = 288
_B = 64
_N = 8192
_F = 64

_info = plsc.get_sparse_core_info()
_NC = _info.num_cores
_NS = _info.num_subcores
_L = _info.num_lanes
_NW = _NC * _NS
_B_PER_W = _B // _NW

_FB = _F // _L
_NCK = 4
_CN = _N // _NCK
_CHUNK = _L * _CN

_mesh = plsc.VectorSubcoreMesh(core_axis_name="c", subcore_axis_name="s")


@functools.partial(
    pl.kernel,
    out_type=jax.ShapeDtypeStruct((_B * _F * _N,), jnp.float32),
    mesh=_mesh,
    scratch_types=[
        pltpu.VMEM((_N,), jnp.float32),
        pltpu.VMEM((_TIME * _F,), jnp.float32),
        pltpu.VMEM((_N,), jnp.int32),
        pltpu.VMEM((2 * _CHUNK,), jnp.float32),
        pltpu.SemaphoreType.DMA,
        pltpu.SemaphoreType.DMA,
    ],
    compiler_params=pltpu.CompilerParams(needs_layout_passes=False),
)
def _emb(xs_hbm, tbl_hbm, out_hbm, xbuf, tbl, idxb, obuf, sem0, sem1):
    wid = lax.axis_index("s") * _NC + lax.axis_index("c")
    pltpu.sync_copy(tbl_hbm, tbl)
    sems = (sem0, sem1)
    lane = lax.iota(jnp.int32, _L)
    foffs = [(lane + d) & (_L - 1) for d in range(_L)]
    soffs = [foffs[d] * _CN + lane for d in range(_L)]

    for j in range(_B_PER_W):
        b = wid * _B_PER_W + j

        pltpu.sync_copy(xs_hbm.at[b], xbuf)

        @plsc.parallel_loop(0, _N, _L, unroll=8)
        def _idx_body(n):
            v = xbuf[pl.ds(n, _L)]
            idxb[pl.ds(n, _L)] = (v * float(_TIME)).astype(jnp.int32) * _F

        def fb_body(fb, carry):
            fbase = fb * _L
            for nc in range(_NCK):
                par = nc % 2
                obase = par * _CHUNK

                def _wait(par=par, obase=obase):
                    pltpu.make_async_copy(
                        out_hbm.at[pl.ds(0, _CHUNK)],
                        obuf.at[pl.ds(obase, _CHUNK)],
                        sems[par],
                    ).wait()

                if j == 0 and nc < 2:
                    pl.when(fb > 0)(_wait)
                else:
                    _wait()

                @plsc.parallel_loop(0, _CN, _L, unroll=4)
                def _gather_body(n0):
                    iv = idxb[pl.ds(nc * _CN + n0, _L)] + fbase
                    sb = obase + n0
                    for d in range(_L):
                        g = plsc.load_gather(tbl, [iv + foffs[d]])
                        plsc.store_scatter(obuf, [soffs[d] + sb], g)

                for fl in range(_L):
                    pltpu.async_copy(
                        obuf.at[pl.ds(obase + fl * _CN, _CN)],
                        out_hbm.at[
                            pl.ds((b * _F + fbase + fl) * _N + nc * _CN, _CN)
                        ],
                        sems[par],
                    )
            return carry

        lax.fori_loop(0, _FB, fb_body, 0)

    for par in range(2):
        pltpu.make_async_copy(
            out_hbm.at[pl.ds(0, _CHUNK)],
            obuf.at[pl.ds(par * _CHUNK, _CHUNK)],
            sems[par],
        ).wait()


def kernel(x, time_day):
    xs = x[:, -1, :, 1]
    out = _emb(xs, time_day.reshape(-1))
    return out.reshape(_B, _F, _N, 1)


# speedup vs baseline: 1.0048x; 1.0048x over previous
import functools

import jax
import jax.numpy as jnp
from jax import lax
from jax.experimental import pallas as pl
from jax.experimental.pallas import tpu as pltpu
from jax.experimental.pallas import tpu_sc as plsc

_TIME = 288
_B = 64
_N = 8192
_F = 64

_info = plsc.get_sparse_core_info()
_NC = _info.num_cores
_NS = _info.num_subcores
_L = _info.num_lanes
_NW = _NC * _NS
_B_PER_W = _B // _NW

_FB = _F // _L
_NCK = 4
_CN = _N // _NCK
_CHUNK = _L * _CN

_mesh = plsc.VectorSubcoreMesh(core_axis_name="c", subcore_axis_name="s")


@functools.partial(
    pl.kernel,
    out_type=jax.ShapeDtypeStruct((_B * _F * _N,), jnp.float32),
    mesh=_mesh,
    scratch_types=[
        pltpu.VMEM((_N,), jnp.float32),
        pltpu.VMEM((_TIME * _F,), jnp.float32),
        pltpu.VMEM((_N,), jnp.int32),
        pltpu.VMEM((2 * _CHUNK,), jnp.float32),
        pltpu.SemaphoreType.DMA,
        pltpu.SemaphoreType.DMA,
    ],
    compiler_params=pltpu.CompilerParams(needs_layout_passes=False),
)
def _emb(xs_hbm, tbl_hbm, out_hbm, xbuf, tbl, idxb, obuf, sem0, sem1):
    wid = lax.axis_index("s") * _NC + lax.axis_index("c")
    pltpu.sync_copy(tbl_hbm, tbl)
    sems = (sem0, sem1)
    lane = lax.iota(jnp.int32, _L)
    foffs = [(lane + d) & (_L - 1) for d in range(_L)]
    soffs = [foffs[d] * _CN + lane for d in range(_L)]

    for j in range(_B_PER_W):
        b = wid * _B_PER_W + j

        pltpu.sync_copy(xs_hbm.at[b], xbuf)

        @plsc.parallel_loop(0, _N, _L, unroll=8)
        def _idx_body(n):
            v = xbuf[pl.ds(n, _L)]
            idxb[pl.ds(n, _L)] = (v * float(_TIME)).astype(jnp.int32) * _F

        def fb_body(fb, carry):
            fbase = fb * _L
            for nc in range(_NCK):
                par = nc % 2
                obase = par * _CHUNK

                def _wait(par=par, obase=obase):
                    pltpu.make_async_copy(
                        out_hbm.at[pl.ds(0, _CHUNK)],
                        obuf.at[pl.ds(obase, _CHUNK)],
                        sems[par],
                    ).wait()

                if j == 0 and nc < 2:
                    pl.when(fb > 0)(_wait)
                else:
                    _wait()

                @plsc.parallel_loop(0, _CN, _L, unroll=2)
                def _gather_body(n0):
                    iv = idxb[pl.ds(nc * _CN + n0, _L)] + fbase
                    sb = obase + n0
                    for d in range(_L):
                        g = plsc.load_gather(tbl, [iv + foffs[d]])
                        plsc.store_scatter(obuf, [soffs[d] + sb], g)

                for fl in range(_L):
                    pltpu.async_copy(
                        obuf.at[pl.ds(obase + fl * _CN, _CN)],
                        out_hbm.at[
                            pl.ds((b * _F + fbase + fl) * _N + nc * _CN, _CN)
                        ],
                        sems[par],
                    )
            return carry

        lax.fori_loop(0, _FB, fb_body, 0)

    for par in range(2):
        pltpu.make_async_copy(
            out_hbm.at[pl.ds(0, _CHUNK)],
            obuf.at[pl.ds(par * _CHUNK, _CHUNK)],
            sems[par],
        ).wait()


def kernel(x, time_day):
    xs = x[:, -1, :, 1]
    out = _emb(xs, time_day.reshape(-1))
    return out.reshape(_B, _F, _N, 1)


# speedup vs baseline: 1.0090x; 1.0041x over previous
import functools

import jax
import jax.numpy as jnp
from jax import lax
from jax.experimental import pallas as pl
from jax.experimental.pallas import tpu as pltpu
from jax.experimental.pallas import tpu_sc as plsc

_TIME = 288
_B = 64
_N = 8192
_F = 64

_info = plsc.get_sparse_core_info()
_NC = _info.num_cores
_NS = _info.num_subcores
_L = _info.num_lanes
_NW = _NC * _NS
_B_PER_W = _B // _NW

_FB = _F // _L
_NCK = 8
_CN = _N // _NCK
_CHUNK = _L * _CN
_NBUF = 4

_mesh = plsc.VectorSubcoreMesh(core_axis_name="c", subcore_axis_name="s")


@functools.partial(
    pl.kernel,
    out_type=jax.ShapeDtypeStruct((_B * _F * _N,), jnp.float32),
    mesh=_mesh,
    scratch_types=[
        pltpu.VMEM((_N,), jnp.float32),
        pltpu.VMEM((_TIME * _F,), jnp.float32),
        pltpu.VMEM((_N,), jnp.int32),
        pltpu.VMEM((_NBUF * _CHUNK,), jnp.float32),
        pltpu.SemaphoreType.DMA,
        pltpu.SemaphoreType.DMA,
        pltpu.SemaphoreType.DMA,
        pltpu.SemaphoreType.DMA,
    ],
    compiler_params=pltpu.CompilerParams(needs_layout_passes=False),
)
def _emb(xs_hbm, tbl_hbm, out_hbm, xbuf, tbl, idxb, obuf, sem0, sem1, sem2, sem3):
    wid = lax.axis_index("s") * _NC + lax.axis_index("c")
    pltpu.sync_copy(tbl_hbm, tbl)
    sems = (sem0, sem1, sem2, sem3)
    lane = lax.iota(jnp.int32, _L)
    foffs = [(lane + d) & (_L - 1) for d in range(_L)]
    soffs = [foffs[d] * _CN + lane for d in range(_L)]

    for j in range(_B_PER_W):
        b = wid * _B_PER_W + j

        pltpu.sync_copy(xs_hbm.at[b], xbuf)

        @plsc.parallel_loop(0, _N, _L, unroll=8)
        def _idx_body(n):
            v = xbuf[pl.ds(n, _L)]
            idxb[pl.ds(n, _L)] = (v * float(_TIME)).astype(jnp.int32) * _F

        def fb_body(fb, carry):
            fbase = fb * _L
            for nc in range(_NCK):
                par = nc % _NBUF
                obase = par * _CHUNK

                def _wait(par=par, obase=obase):
                    pltpu.make_async_copy(
                        out_hbm.at[pl.ds(0, _CHUNK)],
                        obuf.at[pl.ds(obase, _CHUNK)],
                        sems[par],
                    ).wait()

                if j == 0 and nc < _NBUF:
                    pl.when(fb > 0)(_wait)
                else:
                    _wait()

                @plsc.parallel_loop(0, _CN, _L, unroll=2)
                def _gather_body(n0):
                    iv = idxb[pl.ds(nc * _CN + n0, _L)] + fbase
                    sb = obase + n0
                    for d in range(_L):
                        g = plsc.load_gather(tbl, [iv + foffs[d]])
                        plsc.store_scatter(obuf, [soffs[d] + sb], g)

                for fl in range(_L):
                    pltpu.async_copy(
                        obuf.at[pl.ds(obase + fl * _CN, _CN)],
                        out_hbm.at[
                            pl.ds((b * _F + fbase + fl) * _N + nc * _CN, _CN)
                        ],
                        sems[par],
                    )
            return carry

        lax.fori_loop(0, _FB, fb_body, 0)

    for par in range(_NBUF):
        pltpu.make_async_copy(
            out_hbm.at[pl.ds(0, _CHUNK)],
            obuf.at[pl.ds(par * _CHUNK, _CHUNK)],
            sems[par],
        ).wait()


def kernel(x, time_day):
    xs = x[:, -1, :, 1]
    out = _emb(xs, time_day.reshape(-1))
    return out.reshape(_B, _F, _N, 1)
